# baseline (speedup 1.0000x reference)
import functools

import jax
import jax.numpy as jnp
from jax import lax
from jax.experimental import pallas as pl
from jax.experimental.pallas import tpu as pltpu
from jax.experimental.pallas import tpu_sc as plsc

N = 10000
E = 160000
IN_FEATS = 256
H_FEATS = 512

C = 64
B = 80
NTILES = 16
EPT = E // NTILES
NB = EPT // B
NPAD = 10240
NPT = NPAD // NTILES
DPT = NPAD // NTILES
EPTA = 10240
NBB = EPTA // B
GRP = 8
PADDST = 10200

_MESH = plsc.VectorSubcoreMesh(core_axis_name="c", subcore_axis_name="s")


def _zero_vmem(ref, rows, cols):
    zeros16 = jnp.zeros((16,), jnp.float32)

    def body(r, _):
        for j in range(cols // 16):
            ref[r, pl.ds(j * 16, 16)] = zeros16
        return 0

    lax.fori_loop(0, rows, body, 0)


@functools.partial(
    pl.kernel,
    out_type=(
        jax.ShapeDtypeStruct((NPAD,), jnp.float32),
        jax.ShapeDtypeStruct((NPAD,), jnp.float32),
    ),
    mesh=_MESH,
    scratch_types=[
        pltpu.VMEM((NB, B), jnp.int32),
        pltpu.VMEM((B,), jnp.float32),
        pltpu.VMEM((DPT,), jnp.float32),
        pltpu.VMEM_SHARED((NPAD,), jnp.float32),
        pltpu.SemaphoreType.DMA,
    ],
)
def _deg_kernel(src_hbm, dst_hbm, outdeg_hbm, indeg_hbm,
                idx_v, ones_v, zero_v, acc_sh, sem):
    cid = lax.axis_index("c")
    sid = lax.axis_index("s")

    ones16 = jnp.ones((16,), jnp.float32)
    zeros16 = jnp.zeros((16,), jnp.float32)
    for j in range(B // 16):
        ones_v[pl.ds(j * 16, 16)] = ones16

    def zbody(r, _):
        zero_v[pl.ds(r * 16, 16)] = zeros16
        return 0
    lax.fori_loop(0, DPT // 16, zbody, 0)
    pltpu.sync_copy(zero_v, acc_sh.at[pl.ds(sid * DPT, DPT)])

    @pl.when(cid == 0)
    def _():
        pltpu.sync_copy(src_hbm.at[sid], idx_v)

    @pl.when(cid == 1)
    def _():
        pltpu.sync_copy(dst_hbm.at[sid], idx_v)

    plsc.subcore_barrier()

    DWIN = 8

    def abody(b, _):
        pltpu.async_copy(ones_v, acc_sh.at[idx_v.at[b]], sem, add=True)

        @pl.when(b >= DWIN)
        def _():
            pltpu.make_async_copy(
                ones_v, acc_sh.at[idx_v.at[b - DWIN]], sem).wait()
        return 0
    lax.fori_loop(0, NB, abody, 0)

    def dbody(b, _):
        pltpu.make_async_copy(
            ones_v, acc_sh.at[idx_v.at[b]], sem).wait()
        return 0
    lax.fori_loop(NB - DWIN, NB, dbody, 0)

    plsc.subcore_barrier()

    @pl.when(cid == 0)
    def _():
        pltpu.sync_copy(acc_sh.at[pl.ds(sid * DPT, DPT)],
                        outdeg_hbm.at[pl.ds(sid * DPT, DPT)])

    @pl.when(cid == 1)
    def _():
        pltpu.sync_copy(acc_sh.at[pl.ds(sid * DPT, DPT)],
                        indeg_hbm.at[pl.ds(sid * DPT, DPT)])


def _make_agg(nch):
    cps = nch // 2

    @functools.partial(
        pl.kernel,
        out_type=jax.ShapeDtypeStruct((nch, NPAD, C), jnp.float32),
        mesh=_MESH,
        scratch_types=[
            pltpu.VMEM((NBB, B), jnp.int32),
            pltpu.VMEM((NBB, B), jnp.int32),
            pltpu.VMEM((GRP, B), jnp.int32),
            [pltpu.VMEM((B, C), jnp.float32) for _ in range(GRP)],
            pltpu.VMEM_SHARED((NPAD, C), jnp.float32),
            pltpu.SemaphoreType.DMA,
            pltpu.SemaphoreType.DMA,
        ],
        compiler_params=pltpu.CompilerParams(use_tc_tiling_on_sc=False),
    )
    def agg(h_hbm, src_hbm, dst_hbm, out_hbm,
            src_v, dst_v, idx_v, rows_v, acc_sh, gsem, ssem):
        cid = lax.axis_index("c")
        sid = lax.axis_index("s")

        pltpu.sync_copy(src_hbm.at[sid], src_v)
        pltpu.sync_copy(dst_hbm.at[sid], dst_v)

        for cl in range(cps):
            chunk = cid * cps + cl

            _zero_vmem(rows_v[0], B, C)
            for r in range(NPT // B):
                pltpu.sync_copy(
                    rows_v[0], acc_sh.at[pl.ds(sid * NPT + r * B, B)])

            plsc.subcore_barrier()

            def ebody(g, _):
                b0 = g * GRP

                def ibody(j2, _2):
                    for t in range(B // 16):
                        idx_v[j2, pl.ds(t * 16, 16)] = (
                            src_v[b0 + j2, pl.ds(t * 16, 16)] * nch + chunk)
                    return 0
                lax.fori_loop(0, GRP, ibody, 0)

                @pl.when(g > 0)
                def _():
                    for j in range(GRP):
                        pltpu.make_async_copy(
                            rows_v[j], acc_sh.at[dst_v.at[b0 + j]],
                            ssem).wait()

                gds = []
                for j in range(GRP):
                    gds.append(pltpu.async_copy(
                        h_hbm.at[idx_v.at[j]], rows_v[j], gsem))
                for j in range(GRP):
                    gds[j].wait()
                    pltpu.async_copy(
                        rows_v[j], acc_sh.at[dst_v.at[b0 + j]], ssem,
                        add=True)
                return 0
            lax.fori_loop(0, NBB // GRP, ebody, 0)

            for j in range(GRP):
                pltpu.make_async_copy(
                    rows_v[j], acc_sh.at[dst_v.at[j]], ssem).wait()

            plsc.subcore_barrier()

            pltpu.sync_copy(acc_sh.at[pl.ds(sid * NPT, NPT)],
                            out_hbm.at[chunk, pl.ds(sid * NPT, NPT)])

            if cl + 1 < cps:
                plsc.subcore_barrier()

    return agg


_agg2 = _make_agg(4)
_agg4 = _make_agg(8)


MB = 1000


def _prep_body(x_ref, od_ref, id_ref, s_ref, d_ref, xs_ref):
    s = lax.rsqrt(jnp.maximum(od_ref[...], 1.0))
    d = lax.rsqrt(jnp.maximum(id_ref[...], 1.0))
    s_ref[...] = s
    d_ref[...] = d
    xs_ref[...] = x_ref[...] * s


def _prep(x, od_col, id_col):
    return pl.pallas_call(
        _prep_body,
        grid=(N // MB,),
        in_specs=[
            pl.BlockSpec((MB, IN_FEATS), lambda m: (m, 0)),
            pl.BlockSpec((MB, 1), lambda m: (m, 0)),
            pl.BlockSpec((MB, 1), lambda m: (m, 0)),
        ],
        out_specs=[
            pl.BlockSpec((MB, 1), lambda m: (m, 0)),
            pl.BlockSpec((MB, 1), lambda m: (m, 0)),
            pl.BlockSpec((MB, IN_FEATS), lambda m: (m, 0)),
        ],
        out_shape=[
            jax.ShapeDtypeStruct((N, 1), jnp.float32),
            jax.ShapeDtypeStruct((N, 1), jnp.float32),
            jax.ShapeDtypeStruct((N, IN_FEATS), jnp.float32),
        ],
    )(x, od_col, id_col)


def _make_mm(nch, scale):
    def body(a_ref, w_ref, b_ref, d_ref, s_ref, o_ref):
        acc = jnp.dot(a_ref[0], w_ref[pl.ds(0, C), :],
                      preferred_element_type=jnp.float32)
        for c in range(1, nch):
            acc += jnp.dot(a_ref[c], w_ref[pl.ds(c * C, C), :],
                           preferred_element_type=jnp.float32)
        r = jnp.maximum(acc * d_ref[...] + b_ref[...], 0.0)
        if scale:
            r = r * s_ref[...]
        o_ref[...] = r

    def mm(agg, w, b, d_col, s_col):
        return pl.pallas_call(
            body,
            grid=(N // MB,),
            in_specs=[
                pl.BlockSpec((nch, MB, C), lambda m: (0, m, 0)),
                pl.BlockSpec((nch * C, H_FEATS), lambda m: (0, 0)),
                pl.BlockSpec((1, H_FEATS), lambda m: (0, 0)),
                pl.BlockSpec((MB, 1), lambda m: (m, 0)),
                pl.BlockSpec((MB, 1), lambda m: (m, 0)),
            ],
            out_specs=pl.BlockSpec((MB, H_FEATS), lambda m: (m, 0)),
            out_shape=jax.ShapeDtypeStruct((N, H_FEATS), jnp.float32),
        )(agg, w, b, d_col, s_col)

    return mm


_mm2_s = _make_mm(4, True)
_mm4_s = _make_mm(8, True)
_mm4 = _make_mm(8, False)


def kernel(x, edge_index, edge_attr, W0, b0, W1, b1, W2, b2):
    src = edge_index[0]
    dst = edge_index[1]
    src3 = src.reshape(NTILES, NB, B)
    dst3 = dst.reshape(NTILES, NB, B)

    pad = EPTA - EPT
    src_p = jnp.pad(src.reshape(NTILES, EPT),
                    ((0, 0), (0, pad))).reshape(NTILES, NBB, B)
    dst_p = jnp.pad(dst.reshape(NTILES, EPT), ((0, 0), (0, pad)),
                    constant_values=PADDST).reshape(NTILES, NBB, B)

    outdeg_p, indeg_p = _deg_kernel(src3, dst3)
    od_col = outdeg_p[:N].reshape(N, 1)
    id_col = indeg_p[:N].reshape(N, 1)

    s_col, d_col, xs = _prep(x, od_col, id_col)

    agg0 = _agg2(xs.reshape(N * 4, C), src_p, dst_p)
    h1 = _mm2_s(agg0, W0, b0.reshape(1, H_FEATS), d_col, s_col)

    agg1 = _agg4(h1.reshape(N * 8, C), src_p, dst_p)
    h2 = _mm4_s(agg1, W1, b1.reshape(1, H_FEATS), d_col, s_col)

    agg2 = _agg4(h2.reshape(N * 8, C), src_p, dst_p)
    out = _mm4(agg2, W2, b2.reshape(1, H_FEATS), d_col, s_col)
    return out


# speedup vs baseline: 1.7329x; 1.7329x over previous
import functools

import jax
import jax.numpy as jnp
from jax import lax
from jax.experimental import pallas as pl
from jax.experimental.pallas import tpu as pltpu
from jax.experimental.pallas import tpu_sc as plsc

N = 10000
E = 160000
IN_FEATS = 256
H_FEATS = 512

C = 64
B = 80
NTILES = 16
EPT = E // NTILES
NB = EPT // B
NPAD = 10240
NPT = NPAD // NTILES
DPT = NPAD // NTILES
NBB = NB
GRP = 5

_MESH = plsc.VectorSubcoreMesh(core_axis_name="c", subcore_axis_name="s")


def _zero_vmem(ref, rows, cols):
    zeros16 = jnp.zeros((16,), jnp.float32)

    def body(r, _):
        for j in range(cols // 16):
            ref[r, pl.ds(j * 16, 16)] = zeros16
        return 0

    lax.fori_loop(0, rows, body, 0)


@functools.partial(
    pl.kernel,
    out_type=(
        jax.ShapeDtypeStruct((NPAD,), jnp.float32),
        jax.ShapeDtypeStruct((NPAD,), jnp.float32),
    ),
    mesh=_MESH,
    scratch_types=[
        pltpu.VMEM((NB, B), jnp.int32),
        pltpu.VMEM((B,), jnp.float32),
        pltpu.VMEM((DPT,), jnp.float32),
        pltpu.VMEM_SHARED((NPAD,), jnp.float32),
        pltpu.SemaphoreType.DMA,
    ],
)
def _deg_kernel(src_hbm, dst_hbm, outdeg_hbm, indeg_hbm,
                idx_v, ones_v, zero_v, acc_sh, sem):
    cid = lax.axis_index("c")
    sid = lax.axis_index("s")

    ones16 = jnp.ones((16,), jnp.float32)
    zeros16 = jnp.zeros((16,), jnp.float32)
    for j in range(B // 16):
        ones_v[pl.ds(j * 16, 16)] = ones16

    def zbody(r, _):
        zero_v[pl.ds(r * 16, 16)] = zeros16
        return 0
    lax.fori_loop(0, DPT // 16, zbody, 0)
    pltpu.sync_copy(zero_v, acc_sh.at[pl.ds(sid * DPT, DPT)])

    @pl.when(cid == 0)
    def _():
        pltpu.sync_copy(src_hbm.at[sid], idx_v)

    @pl.when(cid == 1)
    def _():
        pltpu.sync_copy(dst_hbm.at[sid], idx_v)

    plsc.subcore_barrier()

    DWIN = 8

    def abody(b, _):
        pltpu.async_copy(ones_v, acc_sh.at[idx_v.at[b]], sem, add=True)

        @pl.when(b >= DWIN)
        def _():
            pltpu.make_async_copy(
                ones_v, acc_sh.at[idx_v.at[b - DWIN]], sem).wait()
        return 0
    lax.fori_loop(0, NB, abody, 0)

    def dbody(b, _):
        pltpu.make_async_copy(
            ones_v, acc_sh.at[idx_v.at[b]], sem).wait()
        return 0
    lax.fori_loop(NB - DWIN, NB, dbody, 0)

    plsc.subcore_barrier()

    @pl.when(cid == 0)
    def _():
        pltpu.sync_copy(acc_sh.at[pl.ds(sid * DPT, DPT)],
                        outdeg_hbm.at[pl.ds(sid * DPT, DPT)])

    @pl.when(cid == 1)
    def _():
        pltpu.sync_copy(acc_sh.at[pl.ds(sid * DPT, DPT)],
                        indeg_hbm.at[pl.ds(sid * DPT, DPT)])


def _make_agg(nch):
    cps = nch // 2

    @functools.partial(
        pl.kernel,
        out_type=jax.ShapeDtypeStruct((nch, NPAD, C), jnp.float32),
        mesh=_MESH,
        scratch_types=[
            pltpu.VMEM((NBB, B), jnp.int32),
            pltpu.VMEM((NBB, B), jnp.int32),
            pltpu.VMEM((GRP, B), jnp.int32),
            [pltpu.VMEM((B, C), jnp.float32) for _ in range(GRP)],
            pltpu.VMEM_SHARED((NPAD, C), jnp.float32),
            pltpu.SemaphoreType.DMA,
            pltpu.SemaphoreType.DMA,
        ],
        compiler_params=pltpu.CompilerParams(use_tc_tiling_on_sc=False),
    )
    def agg(h_hbm, src_hbm, dst_hbm, out_hbm,
            src_v, dst_v, idx_v, rows_v, acc_sh, gsem, ssem):
        cid = lax.axis_index("c")
        sid = lax.axis_index("s")

        pltpu.sync_copy(src_hbm.at[sid], src_v)
        pltpu.sync_copy(dst_hbm.at[sid], dst_v)

        for cl in range(cps):
            chunk = cid * cps + cl

            _zero_vmem(rows_v[0], B, C)
            for r in range(NPT // B):
                pltpu.sync_copy(
                    rows_v[0], acc_sh.at[pl.ds(sid * NPT + r * B, B)])

            plsc.subcore_barrier()

            def ebody(g, _):
                b0 = g * GRP

                def ibody(j2, _2):
                    for t in range(B // 16):
                        idx_v[j2, pl.ds(t * 16, 16)] = (
                            src_v[b0 + j2, pl.ds(t * 16, 16)] * nch + chunk)
                    return 0
                lax.fori_loop(0, GRP, ibody, 0)

                @pl.when(g > 0)
                def _():
                    for j in range(GRP):
                        pltpu.make_async_copy(
                            rows_v[j], acc_sh.at[dst_v.at[b0 + j]],
                            ssem).wait()

                gds = []
                for j in range(GRP):
                    gds.append(pltpu.async_copy(
                        h_hbm.at[idx_v.at[j]], rows_v[j], gsem))
                for j in range(GRP):
                    gds[j].wait()
                    pltpu.async_copy(
                        rows_v[j], acc_sh.at[dst_v.at[b0 + j]], ssem,
                        add=True)
                return 0
            lax.fori_loop(0, NBB // GRP, ebody, 0)

            for j in range(GRP):
                pltpu.make_async_copy(
                    rows_v[j], acc_sh.at[dst_v.at[j]], ssem).wait()

            plsc.subcore_barrier()

            pltpu.sync_copy(acc_sh.at[pl.ds(sid * NPT, NPT)],
                            out_hbm.at[chunk, pl.ds(sid * NPT, NPT)])

            if cl + 1 < cps:
                plsc.subcore_barrier()

    return agg


_agg2 = _make_agg(4)
_agg4 = _make_agg(8)


MB = 1000


def _prep_body(x_ref, od_ref, id_ref, s_ref, d_ref, xs_ref):
    s = lax.rsqrt(jnp.maximum(od_ref[...], 1.0))
    d = lax.rsqrt(jnp.maximum(id_ref[...], 1.0))
    s_ref[...] = s
    d_ref[...] = d
    xs_ref[...] = x_ref[...] * s


def _prep(x, od_col, id_col):
    return pl.pallas_call(
        _prep_body,
        grid=(N // MB,),
        in_specs=[
            pl.BlockSpec((MB, IN_FEATS), lambda m: (m, 0)),
            pl.BlockSpec((MB, 1), lambda m: (m, 0)),
            pl.BlockSpec((MB, 1), lambda m: (m, 0)),
        ],
        out_specs=[
            pl.BlockSpec((MB, 1), lambda m: (m, 0)),
            pl.BlockSpec((MB, 1), lambda m: (m, 0)),
            pl.BlockSpec((MB, IN_FEATS), lambda m: (m, 0)),
        ],
        out_shape=[
            jax.ShapeDtypeStruct((N, 1), jnp.float32),
            jax.ShapeDtypeStruct((N, 1), jnp.float32),
            jax.ShapeDtypeStruct((N, IN_FEATS), jnp.float32),
        ],
    )(x, od_col, id_col)


def _make_mm(nch, scale):
    def body(a_ref, w_ref, b_ref, d_ref, s_ref, o_ref):
        acc = jnp.dot(a_ref[0], w_ref[pl.ds(0, C), :],
                      preferred_element_type=jnp.float32)
        for c in range(1, nch):
            acc += jnp.dot(a_ref[c], w_ref[pl.ds(c * C, C), :],
                           preferred_element_type=jnp.float32)
        r = jnp.maximum(acc * d_ref[...] + b_ref[...], 0.0)
        if scale:
            r = r * s_ref[...]
        o_ref[...] = r

    def mm(agg, w, b, d_col, s_col):
        return pl.pallas_call(
            body,
            grid=(N // MB,),
            in_specs=[
                pl.BlockSpec((nch, MB, C), lambda m: (0, m, 0)),
                pl.BlockSpec((nch * C, H_FEATS), lambda m: (0, 0)),
                pl.BlockSpec((1, H_FEATS), lambda m: (0, 0)),
                pl.BlockSpec((MB, 1), lambda m: (m, 0)),
                pl.BlockSpec((MB, 1), lambda m: (m, 0)),
            ],
            out_specs=pl.BlockSpec((MB, H_FEATS), lambda m: (m, 0)),
            out_shape=jax.ShapeDtypeStruct((N, H_FEATS), jnp.float32),
        )(agg, w, b, d_col, s_col)

    return mm


_mm2_s = _make_mm(4, True)
_mm4_s = _make_mm(8, True)
_mm4 = _make_mm(8, False)


def kernel(x, edge_index, edge_attr, W0, b0, W1, b1, W2, b2):
    src = edge_index[0]
    dst = edge_index[1]
    src3 = src.reshape(NTILES, NB, B)
    dst3 = dst.reshape(NTILES, NB, B)

    outdeg_p, indeg_p = _deg_kernel(src3, dst3)
    od_col = outdeg_p[:N].reshape(N, 1)
    id_col = indeg_p[:N].reshape(N, 1)

    s_col, d_col, xs = _prep(x, od_col, id_col)

    agg0 = _agg2(xs.reshape(N * 4, C), src3, dst3)
    h1 = _mm2_s(agg0, W0, b0.reshape(1, H_FEATS), d_col, s_col)

    agg1 = _agg4(h1.reshape(N * 8, C), src3, dst3)
    h2 = _mm4_s(agg1, W1, b1.reshape(1, H_FEATS), d_col, s_col)

    agg2 = _agg4(h2.reshape(N * 8, C), src3, dst3)
    out = _mm4(agg2, W2, b2.reshape(1, H_FEATS), d_col, s_col)
    return out
